# baseline (speedup 1.0000x reference)
import jax
import jax.numpy as jnp
from jax import lax
from jax.experimental import pallas as pl
from jax.experimental.pallas import tpu as pltpu
from jax.experimental.pallas import tpu_sc as plsc

NUM_CORES = 2
NUM_SUBCORES = 16
NUM_WORKERS = NUM_CORES * NUM_SUBCORES
LANES = 16
CHUNK = 128
D = 64
N_CHUNKS = 50
PAD_CHUNKS = 56


def _max_abs_body(w_ref, o_ref):
    i = pl.program_id(0)
    m = jnp.max(jnp.abs(w_ref[...]))

    @pl.when(i == 0)
    def _():
        o_ref[0, 0] = m

    @pl.when(i != 0)
    def _():
        o_ref[0, 0] = jnp.maximum(o_ref[0, 0], m)


def _max_abs(w2):
    rows, d2 = w2.shape
    grid = 125
    blk = rows // grid
    assert blk * grid == rows
    return pl.pallas_call(
        _max_abs_body,
        grid=(grid,),
        in_specs=[pl.BlockSpec((blk, d2), lambda i: (i, 0))],
        out_specs=pl.BlockSpec(memory_space=pltpu.SMEM),
        out_shape=jax.ShapeDtypeStruct((1, 1), jnp.float32),
    )(w2)


def _gather_quant_body(sup_hbm, par_hbm, table_hbm, maxv_hbm, out_hbm,
                       sup_v, par_v, rows0, rows1, out0, out1, maxv_v,
                       g0, g1, o0, o1):
    sup_pc = CHUNK // 2
    wid = lax.axis_index("s") * NUM_CORES + lax.axis_index("c")
    base = wid * (N_CHUNKS * sup_pc)

    pltpu.sync_copy(sup_hbm.at[wid], sup_v)
    pltpu.sync_copy(par_hbm.at[wid], par_v)
    pltpu.sync_copy(maxv_hbm, maxv_v)
    vmax = maxv_v[...]
    vneg = -vmax
    vthr = vmax * (2.0 ** -24)

    iota = lax.broadcasted_iota(jnp.int32, (LANES,), 0)
    rowsel = iota >> 1
    halfsel = (iota & 1) << 6

    def quantize(j, rows_v, out_v):
        def group_body(g, carry):
            pvec = par_v[j, pl.ds(g * LANES, LANES)]
            src_rows = g * LANES + iota
            dst_rows = g * (LANES // 2) + rowsel

            def col_body(c, carry2):
                w = plsc.load_gather(rows_v, [src_rows, pvec + c])
                q = jnp.where(w > vthr, vmax, vneg)
                plsc.store_scatter(out_v, [dst_rows, halfsel + c], q)
                return carry2

            lax.fori_loop(0, D, col_body, 0, unroll=4)
            return carry

        lax.fori_loop(0, CHUNK // LANES, group_body, 0)

    pltpu.async_copy(table_hbm.at[sup_v.at[0]], rows0, g0)
    pltpu.async_copy(table_hbm.at[sup_v.at[1]], rows1, g1)

    def body(t, carry):
        j0 = 2 * t
        j1 = 2 * t + 1

        pltpu.make_async_copy(table_hbm.at[sup_v.at[j0]], rows0, g0).wait()

        @pl.when(t > 0)
        def _():
            pltpu.make_async_copy(
                out0, out_hbm.at[pl.ds(base, sup_pc)], o0).wait()

        quantize(j0, rows0, out0)
        pltpu.async_copy(out0, out_hbm.at[pl.ds(base + j0 * sup_pc, sup_pc)],
                         o0)

        @pl.when(t < N_CHUNKS // 2 - 1)
        def _():
            pltpu.async_copy(table_hbm.at[sup_v.at[j0 + 2]], rows0, g0)

        pltpu.make_async_copy(table_hbm.at[sup_v.at[j1]], rows1, g1).wait()

        @pl.when(t > 0)
        def _():
            pltpu.make_async_copy(
                out1, out_hbm.at[pl.ds(base, sup_pc)], o1).wait()

        quantize(j1, rows1, out1)
        pltpu.async_copy(out1, out_hbm.at[pl.ds(base + j1 * sup_pc, sup_pc)],
                         o1)

        @pl.when(t < N_CHUNKS // 2 - 1)
        def _():
            pltpu.async_copy(table_hbm.at[sup_v.at[j1 + 2]], rows1, g1)

        return carry

    lax.fori_loop(0, N_CHUNKS // 2, body, 0)
    pltpu.make_async_copy(out0, out_hbm.at[pl.ds(base, sup_pc)], o0).wait()
    pltpu.make_async_copy(out1, out_hbm.at[pl.ds(base, sup_pc)], o1).wait()


def _gather_quant(sup, par, table, maxvec, total):
    d2 = table.shape[1]
    mesh = plsc.VectorSubcoreMesh(core_axis_name="c", subcore_axis_name="s")
    f = pl.kernel(
        _gather_quant_body,
        out_type=jax.ShapeDtypeStruct((total // 2, d2), jnp.float32),
        mesh=mesh,
        scratch_types=[
            pltpu.VMEM((PAD_CHUNKS, CHUNK), jnp.int32),
            pltpu.VMEM((PAD_CHUNKS, CHUNK), jnp.int32),
            pltpu.VMEM((CHUNK, d2), jnp.float32),
            pltpu.VMEM((CHUNK, d2), jnp.float32),
            pltpu.VMEM((CHUNK // 2, d2), jnp.float32),
            pltpu.VMEM((CHUNK // 2, d2), jnp.float32),
            pltpu.VMEM((LANES,), jnp.float32),
            pltpu.SemaphoreType.DMA,
            pltpu.SemaphoreType.DMA,
            pltpu.SemaphoreType.DMA,
            pltpu.SemaphoreType.DMA,
        ],
        compiler_params=pltpu.CompilerParams(
            use_tc_tiling_on_sc=True, needs_layout_passes=False),
    )
    return f(sup, par, table, maxvec)


def kernel(input, weight):
    b, s = input.shape
    total = b * s
    assert N_CHUNKS * CHUNK * NUM_WORKERS == total

    w2 = weight.reshape(weight.shape[0] // 2, 2 * weight.shape[1])
    idx = input.astype(jnp.int32)
    sup3 = (idx >> 1).reshape(NUM_WORKERS, N_CHUNKS, CHUNK)
    par3 = ((idx & 1) << 6).reshape(NUM_WORKERS, N_CHUNKS, CHUNK)
    pad = ((0, 0), (0, PAD_CHUNKS - N_CHUNKS), (0, 0))
    sup3 = jnp.pad(sup3, pad)
    par3 = jnp.pad(par3, pad)

    maxv = _max_abs(w2)
    maxvec = jnp.broadcast_to(maxv.reshape(()), (LANES,))
    out = _gather_quant(sup3, par3, w2, maxvec, total)
    return out.reshape(b, s, weight.shape[1])


# speedup vs baseline: 1.4202x; 1.4202x over previous
import jax
import jax.numpy as jnp
from jax import lax
from jax.experimental import pallas as pl
from jax.experimental.pallas import tpu as pltpu
from jax.experimental.pallas import tpu_sc as plsc

NUM_CORES = 2
NUM_SUBCORES = 16
NUM_WORKERS = NUM_CORES * NUM_SUBCORES
LANES = 16
D = 64
ROWS_PER_TILE = 31250
MAX_CHUNK = 625
B_PER_TILE = 128
SEQ = 50


def _wid():
    return lax.axis_index("s") * NUM_CORES + lax.axis_index("c")


def _max_body(table_hbm, part_hbm, buf0, buf1, acc_v, s0, s1):
    wid = _wid()
    base = wid * ROWS_PER_TILE

    def chunk_start(j, buf, sem):
        pltpu.async_copy(
            table_hbm.at[pl.ds(base + j * MAX_CHUNK, MAX_CHUNK)], buf, sem)

    def chunk_reduce(buf, acc):
        def row_body(r, a):
            for c in range(D // LANES):
                a = jnp.maximum(a, jnp.abs(buf[r, pl.ds(c * LANES, LANES)]))
            return a

        return lax.fori_loop(0, MAX_CHUNK, row_body, acc, unroll=4)

    chunk_start(0, buf0, s0)
    chunk_start(1, buf1, s1)
    n_pairs = ROWS_PER_TILE // MAX_CHUNK // 2

    def body(t, acc):
        pltpu.make_async_copy(
            table_hbm.at[pl.ds(base, MAX_CHUNK)], buf0, s0).wait()
        acc = chunk_reduce(buf0, acc)

        @pl.when(t < n_pairs - 1)
        def _():
            chunk_start(2 * t + 2, buf0, s0)

        pltpu.make_async_copy(
            table_hbm.at[pl.ds(base, MAX_CHUNK)], buf1, s1).wait()
        acc = chunk_reduce(buf1, acc)

        @pl.when(t < n_pairs - 1)
        def _():
            chunk_start(2 * t + 3, buf1, s1)

        return acc

    acc = lax.fori_loop(0, n_pairs, body, jnp.zeros((LANES,), jnp.float32))
    acc_v[...] = acc
    pltpu.sync_copy(acc_v, part_hbm.at[wid])


def _max_partials(weight):
    mesh = plsc.VectorSubcoreMesh(core_axis_name="c", subcore_axis_name="s")
    f = pl.kernel(
        _max_body,
        out_type=jax.ShapeDtypeStruct((NUM_WORKERS, LANES), jnp.float32),
        mesh=mesh,
        scratch_types=[
            pltpu.VMEM((MAX_CHUNK, D), jnp.float32),
            pltpu.VMEM((MAX_CHUNK, D), jnp.float32),
            pltpu.VMEM((LANES,), jnp.float32),
            pltpu.SemaphoreType.DMA,
            pltpu.SemaphoreType.DMA,
        ],
        compiler_params=pltpu.CompilerParams(use_tc_tiling_on_sc=False),
    )
    return f(weight)


def _gather_body(idx_hbm, table_hbm, maxv_hbm, out_hbm,
                 idx_v, maxv_v, rows0, rows1, out0, out1,
                 g0, g1, o0, o1):
    wid = _wid()
    b0 = wid * B_PER_TILE

    pltpu.sync_copy(idx_hbm.at[wid], idx_v)
    pltpu.sync_copy(maxv_hbm, maxv_v)

    vmax = maxv_v[...]
    vneg = -vmax
    vthr = vmax * (2.0 ** -24)

    def quantize(rows_v, out_v):
        def row_body(r, carry):
            for c in range(D // LANES):
                w = rows_v[r, pl.ds(c * LANES, LANES)]
                out_v[r, pl.ds(c * LANES, LANES)] = jnp.where(
                    w > vthr, vmax, vneg)
            return carry

        lax.fori_loop(0, SEQ, row_body, 0, unroll=2)

    pltpu.async_copy(table_hbm.at[idx_v.at[0]], rows0, g0)
    pltpu.async_copy(table_hbm.at[idx_v.at[1]], rows1, g1)
    n_pairs = B_PER_TILE // 2

    def body(t, carry):
        pltpu.make_async_copy(table_hbm.at[idx_v.at[2 * t]], rows0, g0).wait()

        @pl.when(t > 0)
        def _():
            pltpu.make_async_copy(out0, out_hbm.at[b0], o0).wait()

        quantize(rows0, out0)
        pltpu.async_copy(out0, out_hbm.at[b0 + 2 * t], o0)

        @pl.when(t < n_pairs - 1)
        def _():
            pltpu.async_copy(table_hbm.at[idx_v.at[2 * t + 2]], rows0, g0)

        pltpu.make_async_copy(
            table_hbm.at[idx_v.at[2 * t + 1]], rows1, g1).wait()

        @pl.when(t > 0)
        def _():
            pltpu.make_async_copy(out1, out_hbm.at[b0], o1).wait()

        quantize(rows1, out1)
        pltpu.async_copy(out1, out_hbm.at[b0 + 2 * t + 1], o1)

        @pl.when(t < n_pairs - 1)
        def _():
            pltpu.async_copy(table_hbm.at[idx_v.at[2 * t + 3]], rows1, g1)

        return carry

    lax.fori_loop(0, n_pairs, body, 0)
    pltpu.make_async_copy(out0, out_hbm.at[b0], o0).wait()
    pltpu.make_async_copy(out1, out_hbm.at[b0], o1).wait()


def _gather_quant(idx3, weight, maxvec):
    b, s = NUM_WORKERS * B_PER_TILE, SEQ
    mesh = plsc.VectorSubcoreMesh(core_axis_name="c", subcore_axis_name="s")
    f = pl.kernel(
        _gather_body,
        out_type=jax.ShapeDtypeStruct((b, s, D), jnp.float32),
        mesh=mesh,
        scratch_types=[
            pltpu.VMEM((B_PER_TILE, SEQ), jnp.int32),
            pltpu.VMEM((LANES,), jnp.float32),
            pltpu.VMEM((SEQ, D), jnp.float32),
            pltpu.VMEM((SEQ, D), jnp.float32),
            pltpu.VMEM((SEQ, D), jnp.float32),
            pltpu.VMEM((SEQ, D), jnp.float32),
            pltpu.SemaphoreType.DMA,
            pltpu.SemaphoreType.DMA,
            pltpu.SemaphoreType.DMA,
            pltpu.SemaphoreType.DMA,
        ],
        compiler_params=pltpu.CompilerParams(use_tc_tiling_on_sc=False),
    )
    return f(idx3, weight, maxvec)


def kernel(input, weight):
    b, s = input.shape
    assert b == NUM_WORKERS * B_PER_TILE and s == SEQ
    idx3 = input.astype(jnp.int32).reshape(NUM_WORKERS, B_PER_TILE, SEQ)
    partials = _max_partials(weight)
    maxvec = jnp.broadcast_to(jnp.max(partials), (LANES,))
    return _gather_quant(idx3, weight, maxvec)
